# baseline (speedup 1.0000x reference)
import functools
import math

import jax
import jax.numpy as jnp
from jax import lax
from jax.experimental import pallas as pl
from jax.experimental.pallas import tpu as pltpu
from jax.experimental.pallas import tpu_sc as plsc

_VOCAB = 100000
_K = 50048
_DIM = 64
_B = 16384
_X_MAX = 100.0
_ALPHA = 0.75
_LOG_XMAX = math.log(_X_MAX)

_NC, _NS, _L = 2, 16, 16
_NW = _NC * _NS
_BPW = _B // _NW
_CHUNK = 256
_NCHUNK = _BPW // _CHUNK

_W = 128
_GRID = _K // _W


def _pack_body(ca_ref, cb_ref, pa_ref, pb_ref, co_ref, po_ref):
    co_ref[:, 0:_DIM] = ca_ref[...].T
    co_ref[:, _DIM:2 * _DIM] = cb_ref[...].T
    po_ref[:, 0:_DIM] = pa_ref[...].T
    po_ref[:, _DIM:2 * _DIM] = pb_ref[...].T


_pack_call = pl.pallas_call(
    _pack_body,
    grid=(_GRID,),
    in_specs=[
        pl.BlockSpec((_DIM, _W), lambda i: (0, i)),
        pl.BlockSpec((_DIM, _W), lambda i: (0, i + _GRID)),
        pl.BlockSpec((_DIM, _W), lambda i: (0, i)),
        pl.BlockSpec((_DIM, _W), lambda i: (0, i + _GRID)),
    ],
    out_specs=[
        pl.BlockSpec((_W, 2 * _DIM), lambda i: (i, 0)),
        pl.BlockSpec((_W, 2 * _DIM), lambda i: (i, 0)),
    ],
    out_shape=[
        jax.ShapeDtypeStruct((_K, 2 * _DIM), jnp.float32),
        jax.ShapeDtypeStruct((_K, 2 * _DIM), jnp.float32),
    ],
)


_mesh = plsc.VectorSubcoreMesh(core_axis_name="c", subcore_axis_name="s")


@functools.partial(
    pl.kernel,
    out_type=jax.ShapeDtypeStruct((_B,), jnp.float32),
    mesh=_mesh,
    compiler_params=pltpu.CompilerParams(needs_layout_passes=False,
                                         use_tc_tiling_on_sc=True),
    scratch_types=[
        pltpu.VMEM((_BPW,), jnp.int32),
        pltpu.VMEM((_BPW,), jnp.int32),
        pltpu.VMEM((_BPW,), jnp.int32),
        pltpu.VMEM((_BPW,), jnp.int32),
        pltpu.VMEM((_CHUNK, 2 * _DIM), jnp.float32),
        pltpu.VMEM((_CHUNK, 2 * _DIM), jnp.float32),
        pltpu.VMEM((_BPW,), jnp.float32),
        pltpu.VMEM((_BPW,), jnp.float32),
        pltpu.VMEM((_BPW,), jnp.float32),
        pltpu.SemaphoreType.DMA,
    ],
)
def _sc_dot(cidx_hbm, pidx_hbm, cemb_hbm, cbias_hbm, pemb_hbm, pbias_hbm,
            out_hbm, cidx_v, pidx_v, cpack_v, ppack_v, ce_v, pe_v,
            cb_v, pb_v, s_v, sem):
    wid = lax.axis_index("s") * _NC + lax.axis_index("c")
    base = wid * _BPW
    pltpu.sync_copy(cidx_hbm.at[pl.ds(base, _BPW)], cidx_v)
    pltpu.sync_copy(pidx_hbm.at[pl.ds(base, _BPW)], pidx_v)

    for j in range(_BPW // _L):
        cv = cidx_v[pl.ds(j * _L, _L)]
        pv = pidx_v[pl.ds(j * _L, _L)]
        cpack_v[pl.ds(j * _L, _L)] = cv - (cv >= _K).astype(jnp.int32) * _K
        ppack_v[pl.ds(j * _L, _L)] = pv - (pv >= _K).astype(jnp.int32) * _K

    b1 = pltpu.async_copy(cbias_hbm.at[cidx_v], cb_v, sem)
    b2 = pltpu.async_copy(pbias_hbm.at[pidx_v], pb_v, sem)

    iot = lax.iota(jnp.int32, _L)

    for c in range(_NCHUNK):
        g1 = pltpu.async_copy(
            cemb_hbm.at[cpack_v.at[pl.ds(c * _CHUNK, _CHUNK)]], ce_v, sem)
        g2 = pltpu.async_copy(
            pemb_hbm.at[ppack_v.at[pl.ds(c * _CHUNK, _CHUNK)]], pe_v, sem)
        g1.wait()
        g2.wait()
        if c == 0:
            b1.wait()
            b2.wait()

        def group(g, carry, c=c):
            off = c * _CHUNK + g * _L
            rows = g * _L + iot
            ccol0 = (cidx_v[pl.ds(off, _L)] >= _K).astype(jnp.int32) * _DIM
            pcol0 = (pidx_v[pl.ds(off, _L)] >= _K).astype(jnp.int32) * _DIM
            acc0 = cb_v[pl.ds(off, _L)] + pb_v[pl.ds(off, _L)]

            def dstep(d, acc):
                return acc + (plsc.load_gather(ce_v, [rows, ccol0 + d]) *
                              plsc.load_gather(pe_v, [rows, pcol0 + d]))

            acc = lax.fori_loop(0, _DIM, dstep, acc0)
            s_v[pl.ds(off, _L)] = acc
            return carry

        lax.fori_loop(0, _CHUNK // _L, group, 0)

    pltpu.sync_copy(s_v, out_hbm.at[pl.ds(base, _BPW)])


def _loss_body(s_ref, lab_ref, out_ref):
    lab = lab_ref[...]
    ll = jnp.log(lab)
    w = jnp.minimum(jnp.exp(_ALPHA * (ll - _LOG_XMAX)), 1.0)
    diff = s_ref[...] - ll
    out_ref[0, 0] = jnp.sum(w * diff * diff) * (1.0 / _B)


_loss_call = pl.pallas_call(
    _loss_body,
    out_shape=jax.ShapeDtypeStruct((1, 1), jnp.float32),
    in_specs=[
        pl.BlockSpec(memory_space=pltpu.VMEM),
        pl.BlockSpec(memory_space=pltpu.VMEM),
    ],
    out_specs=pl.BlockSpec(memory_space=pltpu.SMEM),
)


def kernel(c_data, p_data, labels, c_embed, c_bias, p_embed, p_bias):
    ce2, pe2 = _pack_call(c_embed.T, c_embed.T, p_embed.T, p_embed.T)
    s = _sc_dot(c_data.astype(jnp.int32), p_data.astype(jnp.int32),
                ce2, c_bias.reshape(-1), pe2, p_bias.reshape(-1))
    out = _loss_call(s.reshape(128, 128), labels.reshape(128, 128))
    return out[0, 0]


# speedup vs baseline: 1.7339x; 1.7339x over previous
import functools
import math

import jax
import jax.numpy as jnp
from jax import lax
from jax.experimental import pallas as pl
from jax.experimental.pallas import tpu as pltpu
from jax.experimental.pallas import tpu_sc as plsc

_VOCAB = 100000
_DIM = 64
_B = 16384
_X_MAX = 100.0
_ALPHA = 0.75
_LOG_XMAX = math.log(_X_MAX)

_NC, _NS, _L = 2, 16, 16
_NW = _NC * _NS
_BPW = _B // _NW
_CHUNK = 256
_NCHUNK = _BPW // _CHUNK


_mesh = plsc.VectorSubcoreMesh(core_axis_name="c", subcore_axis_name="s")


@functools.partial(
    pl.kernel,
    out_type=jax.ShapeDtypeStruct((_B,), jnp.float32),
    mesh=_mesh,
    compiler_params=pltpu.CompilerParams(needs_layout_passes=False,
                                         use_tc_tiling_on_sc=True),
    scratch_types=[
        pltpu.VMEM((_BPW,), jnp.int32),
        pltpu.VMEM((_BPW,), jnp.int32),
        pltpu.VMEM((_BPW,), jnp.int32),
        pltpu.VMEM((_BPW,), jnp.int32),
        pltpu.VMEM((_CHUNK, 2 * _DIM), jnp.float32),
        pltpu.VMEM((_CHUNK, 2 * _DIM), jnp.float32),
        pltpu.VMEM((_BPW,), jnp.float32),
        pltpu.VMEM((_BPW,), jnp.float32),
        pltpu.VMEM((_BPW,), jnp.float32),
        pltpu.SemaphoreType.DMA,
    ],
)
def _sc_dot(cidx_hbm, pidx_hbm, cemb_hbm, cbias_hbm, pemb_hbm, pbias_hbm,
            out_hbm, cidx_v, pidx_v, cpack_v, ppack_v, ce_v, pe_v,
            cb_v, pb_v, s_v, sem):
    wid = lax.axis_index("s") * _NC + lax.axis_index("c")
    base = wid * _BPW
    pltpu.sync_copy(cidx_hbm.at[pl.ds(base, _BPW)], cidx_v)
    pltpu.sync_copy(pidx_hbm.at[pl.ds(base, _BPW)], pidx_v)

    for j in range(_BPW // _L):
        cv = cidx_v[pl.ds(j * _L, _L)]
        pv = pidx_v[pl.ds(j * _L, _L)]
        cpack_v[pl.ds(j * _L, _L)] = lax.shift_right_logical(cv, 1)
        ppack_v[pl.ds(j * _L, _L)] = lax.shift_right_logical(pv, 1)

    b1 = pltpu.async_copy(cbias_hbm.at[cidx_v], cb_v, sem)
    b2 = pltpu.async_copy(pbias_hbm.at[pidx_v], pb_v, sem)

    iot = lax.iota(jnp.int32, _L)

    for c in range(_NCHUNK):
        g1 = pltpu.async_copy(
            cemb_hbm.at[cpack_v.at[pl.ds(c * _CHUNK, _CHUNK)]], ce_v, sem)
        g2 = pltpu.async_copy(
            pemb_hbm.at[ppack_v.at[pl.ds(c * _CHUNK, _CHUNK)]], pe_v, sem)
        g1.wait()
        g2.wait()
        if c == 0:
            b1.wait()
            b2.wait()

        def group(g, carry, c=c):
            off = c * _CHUNK + g * _L
            rows = g * _L + iot
            ccol0 = (cidx_v[pl.ds(off, _L)] & 1) * _DIM
            pcol0 = (pidx_v[pl.ds(off, _L)] & 1) * _DIM
            acc0 = cb_v[pl.ds(off, _L)] + pb_v[pl.ds(off, _L)]

            def dstep(d, acc):
                return acc + (plsc.load_gather(ce_v, [rows, ccol0 + d]) *
                              plsc.load_gather(pe_v, [rows, pcol0 + d]))

            acc = lax.fori_loop(0, _DIM, dstep, acc0)
            s_v[pl.ds(off, _L)] = acc
            return carry

        lax.fori_loop(0, _CHUNK // _L, group, 0)

    pltpu.sync_copy(s_v, out_hbm.at[pl.ds(base, _BPW)])


def _loss_body(s_ref, lab_ref, out_ref):
    lab = lab_ref[...]
    ll = jnp.log(lab)
    w = jnp.minimum(jnp.exp(_ALPHA * (ll - _LOG_XMAX)), 1.0)
    diff = s_ref[...] - ll
    out_ref[0, 0] = jnp.sum(w * diff * diff) * (1.0 / _B)


_loss_call = pl.pallas_call(
    _loss_body,
    out_shape=jax.ShapeDtypeStruct((1, 1), jnp.float32),
    in_specs=[
        pl.BlockSpec(memory_space=pltpu.VMEM),
        pl.BlockSpec(memory_space=pltpu.VMEM),
    ],
    out_specs=pl.BlockSpec(memory_space=pltpu.SMEM),
)


def kernel(c_data, p_data, labels, c_embed, c_bias, p_embed, p_bias):
    ce2 = c_embed.reshape(_VOCAB // 2, 2 * _DIM)
    pe2 = p_embed.reshape(_VOCAB // 2, 2 * _DIM)
    s = _sc_dot(c_data.astype(jnp.int32), p_data.astype(jnp.int32),
                ce2, c_bias.reshape(-1), pe2, p_bias.reshape(-1))
    out = _loss_call(s.reshape(128, 128), labels.reshape(128, 128))
    return out[0, 0]


# speedup vs baseline: 1.9373x; 1.1173x over previous
import functools
import math

import jax
import jax.numpy as jnp
from jax import lax
from jax.experimental import pallas as pl
from jax.experimental.pallas import tpu as pltpu
from jax.experimental.pallas import tpu_sc as plsc

_VOCAB = 100000
_DIM = 64
_B = 16384
_X_MAX = 100.0
_ALPHA = 0.75
_LOG_XMAX = math.log(_X_MAX)

_NC, _NS, _L = 2, 16, 16
_DPS = _DIM // _NS
_OCHUNK = 4096
_BCHUNK = _B // _NS


_mesh = plsc.VectorSubcoreMesh(core_axis_name="c", subcore_axis_name="s")


@functools.partial(
    pl.kernel,
    out_type=[
        jax.ShapeDtypeStruct((_DIM, _B), jnp.float32),
        jax.ShapeDtypeStruct((_DIM, _B), jnp.float32),
        jax.ShapeDtypeStruct((_B,), jnp.float32),
        jax.ShapeDtypeStruct((_B,), jnp.float32),
    ],
    mesh=_mesh,
    compiler_params=pltpu.CompilerParams(needs_layout_passes=False,
                                         use_tc_tiling_on_sc=False),
    scratch_types=[
        pltpu.VMEM((_B,), jnp.int32),
        pltpu.VMEM((_VOCAB,), jnp.float32),
        pltpu.VMEM((_OCHUNK,), jnp.float32),
        pltpu.VMEM((_BCHUNK,), jnp.float32),
        pltpu.SemaphoreType.DMA,
    ],
)
def _sc_extract(cidx_hbm, pidx_hbm, cembT_hbm, cbias_hbm, pembT_hbm,
                pbias_hbm, ceT_hbm, peT_hbm, cbg_hbm, pbg_hbm,
                idx_v, row_v, out_v, bias_v, sem):
    core = lax.axis_index("c")
    sub = lax.axis_index("s")

    def do_table(idx_hbm, embT_hbm, bias_hbm, outT_hbm, bg_hbm):
        pltpu.sync_copy(idx_hbm, idx_v)

        bsl = pl.ds(sub * _BCHUNK, _BCHUNK)
        pltpu.async_copy(bias_hbm.at[idx_v.at[bsl]], bias_v, sem).wait()
        pltpu.sync_copy(bias_v, bg_hbm.at[bsl])

        for k in range(_DPS):
            d = sub * _DPS + k
            pltpu.sync_copy(embT_hbm.at[d], row_v)
            for q in range(_B // _OCHUNK):

                def step(j, carry, q=q):
                    for u in range(4):
                        off = j * (4 * _L) + u * _L
                        idx16 = idx_v[pl.ds(q * _OCHUNK + off, _L)]
                        out_v[pl.ds(off, _L)] = plsc.load_gather(
                            row_v, [idx16])
                    return carry

                lax.fori_loop(0, _OCHUNK // (4 * _L), step, 0)
                pltpu.sync_copy(out_v,
                                outT_hbm.at[d, pl.ds(q * _OCHUNK, _OCHUNK)])

    @pl.when(core == 0)
    def _():
        do_table(cidx_hbm, cembT_hbm, cbias_hbm, ceT_hbm, cbg_hbm)

    @pl.when(core == 1)
    def _():
        do_table(pidx_hbm, pembT_hbm, pbias_hbm, peT_hbm, pbg_hbm)


def _loss_body(ceT_ref, peT_ref, cb_ref, pb_ref, lab_ref, out_ref):
    s = jnp.sum(ceT_ref[...] * peT_ref[...], axis=0)
    lab = lab_ref[...]
    ll = jnp.log(lab)
    w = jnp.minimum(jnp.exp(_ALPHA * (ll - _LOG_XMAX)), 1.0)
    diff = s + cb_ref[...] + pb_ref[...] - ll
    out_ref[0, 0] = jnp.sum(w * diff * diff) * (1.0 / _B)


_loss_call = pl.pallas_call(
    _loss_body,
    out_shape=jax.ShapeDtypeStruct((1, 1), jnp.float32),
    in_specs=[
        pl.BlockSpec(memory_space=pltpu.VMEM),
        pl.BlockSpec(memory_space=pltpu.VMEM),
        pl.BlockSpec(memory_space=pltpu.VMEM),
        pl.BlockSpec(memory_space=pltpu.VMEM),
        pl.BlockSpec(memory_space=pltpu.VMEM),
    ],
    out_specs=pl.BlockSpec(memory_space=pltpu.SMEM),
)


def kernel(c_data, p_data, labels, c_embed, c_bias, p_embed, p_bias):
    ceT, peT, cbg, pbg = _sc_extract(
        c_data.astype(jnp.int32), p_data.astype(jnp.int32),
        c_embed.T, c_bias.reshape(-1), p_embed.T, p_bias.reshape(-1))
    out = _loss_call(ceT, peT, cbg, pbg, labels)
    return out[0, 0]
